# baseline (speedup 1.0000x reference)
import functools

import jax
import jax.numpy as jnp
from jax import lax
from jax.experimental import pallas as pl
from jax.experimental.pallas import tpu as pltpu
from jax.experimental.pallas import tpu_sc as plsc

NC = 2
NS = 16
NW = NC * NS
L = 16

B = 4096
F = 26
D = 16

BPW = B // NW
NG = BPW // L


def _fm_body(xt_hbm, embt_hbm, lint_hbm, bias_hbm, out_hbm,
             idxt_v, cols_v, lin_v, ix_v, out_v, bias_v, sem_e, sem_l):
    c = lax.axis_index("c")
    s = lax.axis_index("s")
    wid = s * NC + c
    base = wid * BPW

    def stage(f, carry):
        pltpu.sync_copy(xt_hbm.at[f, pl.ds(base, BPW)], idxt_v.at[f])
        return carry

    lax.fori_loop(0, F, stage, 0)
    pltpu.sync_copy(bias_hbm, bias_v)

    def fire_lin(f, carry):
        pltpu.make_async_copy(
            lint_hbm.at[0].at[idxt_v.at[f]], lin_v.at[f], sem_l,
        ).start()
        return carry

    lax.fori_loop(0, F, fire_lin, 0)

    def fire_emb(k, carry):
        d = k // F
        f = k - d * F
        pltpu.make_async_copy(
            embt_hbm.at[d].at[idxt_v.at[f]], cols_v.at[k], sem_e,
        ).start()
        return carry

    lax.fori_loop(0, D * F, fire_emb, 0)

    def init_ix(g, carry):
        ix_v[pl.ds(g * L, L)] = jnp.zeros((L,), jnp.float32)
        return carry

    lax.fori_loop(0, NG, init_ix, 0)

    def per_d(d, carry):
        def drain(f, c2):
            pltpu.make_async_copy(
                embt_hbm.at[0, pl.ds(0, BPW)], cols_v.at[d * F + f], sem_e,
            ).wait()
            return c2

        lax.fori_loop(0, F, drain, 0)

        def per_group(g, c2):
            col = pl.ds(g * L, L)
            v = cols_v[d * F, col]
            s_acc = v
            ss_acc = v * v
            for f in range(1, F):
                v = cols_v[d * F + f, col]
                s_acc = s_acc + v
                ss_acc = ss_acc + v * v
            ix_v[col] = ix_v[col] + s_acc * s_acc - ss_acc
            return c2

        lax.fori_loop(0, NG, per_group, 0)
        return carry

    lax.fori_loop(0, D, per_d, 0)

    def drain_lin(f, carry):
        pltpu.make_async_copy(
            lint_hbm.at[0, pl.ds(0, BPW)], lin_v.at[f], sem_l,
        ).wait()
        return carry

    lax.fori_loop(0, F, drain_lin, 0)
    bias_vec = bias_v[...]

    def finish(g, carry):
        col = pl.ds(g * L, L)
        lin_acc = lin_v[0, col]
        for f in range(1, F):
            lin_acc = lin_acc + lin_v[f, col]
        z = ix_v[col] + lin_acc + bias_vec
        out_v[col] = 1.0 / (1.0 + jnp.exp(-z))
        return carry

    lax.fori_loop(0, NG, finish, 0)

    pltpu.sync_copy(out_v, out_hbm.at[pl.ds(base, BPW)])


@functools.partial(
    pl.kernel,
    out_type=jax.ShapeDtypeStruct((B,), jnp.float32),
    mesh=plsc.VectorSubcoreMesh(core_axis_name="c", subcore_axis_name="s"),
    scratch_types=[
        pltpu.VMEM((F, BPW), jnp.int32),
        pltpu.VMEM((D * F, BPW), jnp.float32),
        pltpu.VMEM((F, BPW), jnp.float32),
        pltpu.VMEM((BPW,), jnp.float32),
        pltpu.VMEM((BPW,), jnp.float32),
        pltpu.VMEM((L,), jnp.float32),
        pltpu.SemaphoreType.DMA,
        pltpu.SemaphoreType.DMA,
    ],
    compiler_params=pltpu.CompilerParams(use_tc_tiling_on_sc=False),
)
def _fm_kernel(xt_hbm, embt_hbm, lint_hbm, bias_hbm, out_hbm,
               idxt_v, cols_v, lin_v, ix_v, out_v, bias_v, sem_e, sem_l):
    _fm_body(xt_hbm, embt_hbm, lint_hbm, bias_hbm, out_hbm,
             idxt_v, cols_v, lin_v, ix_v, out_v, bias_v, sem_e, sem_l)


def kernel(x, linear_w, emb_w, bias):
    xt = x.astype(jnp.int32).T
    embt = emb_w.T
    lint = linear_w.T
    bias_vec = jnp.broadcast_to(bias.astype(jnp.float32), (L,))
    out = _fm_kernel(xt, embt, lint, bias_vec)
    return out.reshape(B, 1)


# speedup vs baseline: 1.0306x; 1.0306x over previous
import functools

import jax
import jax.numpy as jnp
from jax import lax
from jax.experimental import pallas as pl
from jax.experimental.pallas import tpu as pltpu
from jax.experimental.pallas import tpu_sc as plsc

NC = 2
NS = 16
NW = NC * NS
L = 16

B = 4096
F = 26
D = 16

BPW = B // NW
NG = BPW // L


def _fm_body(xt_hbm, embt_hbm, lint_hbm, bias_hbm, out_hbm,
             idxt_v, cols_v, lin_v, ix_v, out_v, bias_v, sem_e, sem_l):
    c = lax.axis_index("c")
    s = lax.axis_index("s")
    wid = s * NC + c
    base = wid * BPW

    def stage(f, carry):
        pltpu.sync_copy(xt_hbm.at[f, wid], idxt_v.at[f])
        return carry

    lax.fori_loop(0, F, stage, 0)
    pltpu.sync_copy(bias_hbm, bias_v)

    def fire_lin(f, carry):
        pltpu.make_async_copy(
            lint_hbm.at[idxt_v.at[f]], lin_v.at[f], sem_l,
        ).start()
        return carry

    lax.fori_loop(0, F, fire_lin, 0)

    def fire_emb(k, carry):
        d = k // F
        f = k - d * F
        pltpu.make_async_copy(
            embt_hbm.at[d].at[idxt_v.at[f]], cols_v.at[k], sem_e,
        ).start()
        return carry

    lax.fori_loop(0, D * F, fire_emb, 0)

    def init_ix(g, carry):
        ix_v[pl.ds(g * L, L)] = jnp.zeros((L,), jnp.float32)
        return carry

    lax.fori_loop(0, NG, init_ix, 0)

    def per_d(d, carry):
        def drain(f, c2):
            pltpu.make_async_copy(
                embt_hbm.at[0, pl.ds(0, BPW)], cols_v.at[d * F + f], sem_e,
            ).wait()
            return c2

        lax.fori_loop(0, F, drain, 0)

        def per_group(g, c2):
            col = pl.ds(g * L, L)
            v = cols_v[d * F, col]
            s_acc = v
            ss_acc = v * v
            for f in range(1, F):
                v = cols_v[d * F + f, col]
                s_acc = s_acc + v
                ss_acc = ss_acc + v * v
            ix_v[col] = ix_v[col] + s_acc * s_acc - ss_acc
            return c2

        lax.fori_loop(0, NG, per_group, 0)
        return carry

    lax.fori_loop(0, D, per_d, 0)

    def drain_lin(f, carry):
        pltpu.make_async_copy(
            lint_hbm.at[pl.ds(0, BPW)], lin_v.at[f], sem_l,
        ).wait()
        return carry

    lax.fori_loop(0, F, drain_lin, 0)
    bias_vec = bias_v[...]

    def finish(g, carry):
        col = pl.ds(g * L, L)
        lin_acc = lin_v[0, col]
        for f in range(1, F):
            lin_acc = lin_acc + lin_v[f, col]
        z = ix_v[col] + lin_acc + bias_vec
        out_v[col] = 1.0 / (1.0 + jnp.exp(-z))
        return carry

    lax.fori_loop(0, NG, finish, 0)

    pltpu.sync_copy(out_v, out_hbm.at[pl.ds(base, BPW)])


@functools.partial(
    pl.kernel,
    out_type=jax.ShapeDtypeStruct((B,), jnp.float32),
    mesh=plsc.VectorSubcoreMesh(core_axis_name="c", subcore_axis_name="s"),
    scratch_types=[
        pltpu.VMEM((F, BPW), jnp.int32),
        pltpu.VMEM((D * F, BPW), jnp.float32),
        pltpu.VMEM((F, BPW), jnp.float32),
        pltpu.VMEM((BPW,), jnp.float32),
        pltpu.VMEM((BPW,), jnp.float32),
        pltpu.VMEM((L,), jnp.float32),
        pltpu.SemaphoreType.DMA,
        pltpu.SemaphoreType.DMA,
    ],
    compiler_params=pltpu.CompilerParams(use_tc_tiling_on_sc=False),
)
def _fm_kernel(xt_hbm, embt_hbm, lint_hbm, bias_hbm, out_hbm,
               idxt_v, cols_v, lin_v, ix_v, out_v, bias_v, sem_e, sem_l):
    _fm_body(xt_hbm, embt_hbm, lint_hbm, bias_hbm, out_hbm,
             idxt_v, cols_v, lin_v, ix_v, out_v, bias_v, sem_e, sem_l)


def kernel(x, linear_w, emb_w, bias):
    xt = x.astype(jnp.int32).T.reshape(F, NW, BPW)
    embt = emb_w.T
    lint = linear_w.reshape(-1)
    bias_vec = jnp.broadcast_to(bias.astype(jnp.float32), (L,))
    out = _fm_kernel(xt, embt, lint, bias_vec)
    return out.reshape(B, 1)


# speedup vs baseline: 3.0468x; 2.9563x over previous
import functools

import jax
import jax.numpy as jnp
from jax import lax
from jax.experimental import pallas as pl
from jax.experimental.pallas import tpu as pltpu
from jax.experimental.pallas import tpu_sc as plsc

NC = 2
NS = 16
NW = NC * NS
L = 16

B = 4096
F = 26
D = 16
V = 1_000_000

BPW = B // NW
NG = BPW // L
CT = 7813
VPAD = CT * 128
RCH = 13


def _relayout_body(in_ref, out_ref):
    x = in_ref[...].reshape(D, RCH, 128)
    out_ref[...] = x.transpose(1, 0, 2)


_relayout = pl.pallas_call(
    _relayout_body,
    grid=(CT // RCH,),
    in_specs=[pl.BlockSpec((D, RCH * 128), lambda c: (0, c))],
    out_specs=pl.BlockSpec((RCH, D, 128), lambda c: (c, 0, 0)),
    out_shape=jax.ShapeDtypeStruct((CT, D, 128), jnp.float32),
)


def _fm_body(xt_hbm, embf_hbm, lint_hbm, bias_hbm, out_hbm,
             idxt_v, addr_v, cols_v, lin_v, ix_v, out_v, bias_v,
             sem_e, sem_l):
    c = lax.axis_index("c")
    s = lax.axis_index("s")
    wid = s * NC + c
    base = wid * BPW

    def stage(f, carry):
        pltpu.sync_copy(xt_hbm.at[f, wid], idxt_v.at[f])
        return carry

    lax.fori_loop(0, F, stage, 0)
    pltpu.sync_copy(bias_hbm, bias_v)

    def fire_lin(f, carry):
        pltpu.make_async_copy(
            lint_hbm.at[idxt_v.at[f]], lin_v.at[f], sem_l,
        ).start()
        return carry

    lax.fori_loop(0, F, fire_lin, 0)

    def mk_addr(k, carry):
        f = k // (BPW // L)
        j = k - f * (BPW // L)
        sl = pl.ds(j * L, L)
        v = idxt_v[f, sl]
        b = ((v >> 7) << 11) | (v & 127)
        for d in range(D):
            addr_v[d * F + f, sl] = b + (d * 128)
        return carry

    lax.fori_loop(0, F * (BPW // L), mk_addr, 0)

    def fire_emb(k, carry):
        pltpu.make_async_copy(
            embf_hbm.at[addr_v.at[k]], cols_v.at[k], sem_e,
        ).start()
        return carry

    lax.fori_loop(0, D * F, fire_emb, 0)

    def init_ix(g, carry):
        ix_v[pl.ds(g * L, L)] = jnp.zeros((L,), jnp.float32)
        return carry

    lax.fori_loop(0, NG, init_ix, 0)

    def per_d(d, carry):
        def drain(f, c2):
            pltpu.make_async_copy(
                embf_hbm.at[pl.ds(0, BPW)], cols_v.at[d * F + f], sem_e,
            ).wait()
            return c2

        lax.fori_loop(0, F, drain, 0)

        def per_group(g, c2):
            col = pl.ds(g * L, L)
            v = cols_v[d * F, col]
            s_acc = v
            ss_acc = v * v
            for f in range(1, F):
                v = cols_v[d * F + f, col]
                s_acc = s_acc + v
                ss_acc = ss_acc + v * v
            ix_v[col] = ix_v[col] + s_acc * s_acc - ss_acc
            return c2

        lax.fori_loop(0, NG, per_group, 0)
        return carry

    lax.fori_loop(0, D, per_d, 0)

    def drain_lin(f, carry):
        pltpu.make_async_copy(
            lint_hbm.at[pl.ds(0, BPW)], lin_v.at[f], sem_l,
        ).wait()
        return carry

    lax.fori_loop(0, F, drain_lin, 0)
    bias_vec = bias_v[...]

    def finish(g, carry):
        col = pl.ds(g * L, L)
        lin_acc = lin_v[0, col]
        for f in range(1, F):
            lin_acc = lin_acc + lin_v[f, col]
        z = ix_v[col] + lin_acc + bias_vec
        out_v[col] = 1.0 / (1.0 + jnp.exp(-z))
        return carry

    lax.fori_loop(0, NG, finish, 0)

    pltpu.sync_copy(out_v, out_hbm.at[pl.ds(base, BPW)])


@functools.partial(
    pl.kernel,
    out_type=jax.ShapeDtypeStruct((B,), jnp.float32),
    mesh=plsc.VectorSubcoreMesh(core_axis_name="c", subcore_axis_name="s"),
    scratch_types=[
        pltpu.VMEM((F, BPW), jnp.int32),
        pltpu.VMEM((D * F, BPW), jnp.int32),
        pltpu.VMEM((D * F, BPW), jnp.float32),
        pltpu.VMEM((F, BPW), jnp.float32),
        pltpu.VMEM((BPW,), jnp.float32),
        pltpu.VMEM((BPW,), jnp.float32),
        pltpu.VMEM((L,), jnp.float32),
        pltpu.SemaphoreType.DMA,
        pltpu.SemaphoreType.DMA,
    ],
    compiler_params=pltpu.CompilerParams(use_tc_tiling_on_sc=False),
)
def _fm_kernel(xt_hbm, embf_hbm, lint_hbm, bias_hbm, out_hbm,
               idxt_v, addr_v, cols_v, lin_v, ix_v, out_v, bias_v,
               sem_e, sem_l):
    _fm_body(xt_hbm, embf_hbm, lint_hbm, bias_hbm, out_hbm,
             idxt_v, addr_v, cols_v, lin_v, ix_v, out_v, bias_v,
             sem_e, sem_l)


def kernel(x, linear_w, emb_w, bias):
    xt = x.astype(jnp.int32).T.reshape(F, NW, BPW)
    embf = _relayout(emb_w.T).reshape(CT * D * 128)
    lint = jnp.pad(linear_w, ((0, VPAD - V), (0, 0))).T.reshape(VPAD)
    bias_vec = jnp.broadcast_to(bias.astype(jnp.float32), (L,))
    out = _fm_kernel(xt, embf, lint, bias_vec)
    return out.reshape(B, 1)


# speedup vs baseline: 3.0924x; 1.0150x over previous
import functools

import jax
import jax.numpy as jnp
from jax import lax
from jax.experimental import pallas as pl
from jax.experimental.pallas import tpu as pltpu
from jax.experimental.pallas import tpu_sc as plsc

NC = 2
NS = 16
NW = NC * NS
L = 16

B = 4096
F = 26
D = 16
V = 1_000_000

BPW = B // NW
NG = BPW // L
CT = 7813
VPAD = CT * 128
RCH = 13


def _relayout_body(in_ref, out_ref):
    for j in range(RCH):
        out_ref[j] = in_ref[:, pl.ds(j * 128, 128)]


_relayout = pl.pallas_call(
    _relayout_body,
    grid=(CT // RCH,),
    in_specs=[pl.BlockSpec((D, RCH * 128), lambda c: (0, c))],
    out_specs=pl.BlockSpec((RCH, D, 128), lambda c: (c, 0, 0)),
    out_shape=jax.ShapeDtypeStruct((CT, D, 128), jnp.float32),
)


def _fm_body(xt_hbm, embf_hbm, lint_hbm, bias_hbm, out_hbm,
             idxt_v, addr_v, cols_v, lin_v, ix_v, out_v, bias_v,
             sem_e, sem_l):
    c = lax.axis_index("c")
    s = lax.axis_index("s")
    wid = s * NC + c
    base = wid * BPW

    def stage(f, carry):
        pltpu.sync_copy(xt_hbm.at[f, wid], idxt_v.at[f])
        return carry

    lax.fori_loop(0, F, stage, 0)
    pltpu.sync_copy(bias_hbm, bias_v)

    def fire_lin(f, carry):
        pltpu.make_async_copy(
            lint_hbm.at[idxt_v.at[f]], lin_v.at[f], sem_l,
        ).start()
        return carry

    lax.fori_loop(0, F, fire_lin, 0)

    def mk_addr(k, carry):
        f = k // (BPW // L)
        j = k - f * (BPW // L)
        sl = pl.ds(j * L, L)
        v = idxt_v[f, sl]
        b = ((v >> 7) << 11) | (v & 127)
        for d in range(D):
            addr_v[d * F + f, sl] = b + (d * 128)
        return carry

    lax.fori_loop(0, F * (BPW // L), mk_addr, 0)

    def fire_emb(k, carry):
        pltpu.make_async_copy(
            embf_hbm.at[addr_v.at[k]], cols_v.at[k], sem_e,
        ).start()
        return carry

    lax.fori_loop(0, D * F, fire_emb, 0)

    def init_ix(g, carry):
        ix_v[pl.ds(g * L, L)] = jnp.zeros((L,), jnp.float32)
        return carry

    lax.fori_loop(0, NG, init_ix, 0)

    def per_d(d, carry):
        def drain(f, c2):
            pltpu.make_async_copy(
                embf_hbm.at[pl.ds(0, BPW)], cols_v.at[d * F + f], sem_e,
            ).wait()
            return c2

        lax.fori_loop(0, F, drain, 0)

        def per_group(g, c2):
            col = pl.ds(g * L, L)
            v = cols_v[d * F, col]
            s_acc = v
            ss_acc = v * v
            for f in range(1, F):
                v = cols_v[d * F + f, col]
                s_acc = s_acc + v
                ss_acc = ss_acc + v * v
            ix_v[col] = ix_v[col] + s_acc * s_acc - ss_acc
            return c2

        lax.fori_loop(0, NG, per_group, 0)
        return carry

    lax.fori_loop(0, D, per_d, 0)

    def drain_lin(f, carry):
        pltpu.make_async_copy(
            lint_hbm.at[pl.ds(0, BPW)], lin_v.at[f], sem_l,
        ).wait()
        return carry

    lax.fori_loop(0, F, drain_lin, 0)
    bias_vec = bias_v[...]

    def finish(g, carry):
        col = pl.ds(g * L, L)
        lin_acc = lin_v[0, col]
        for f in range(1, F):
            lin_acc = lin_acc + lin_v[f, col]
        z = ix_v[col] + lin_acc + bias_vec
        out_v[col] = 1.0 / (1.0 + jnp.exp(-z))
        return carry

    lax.fori_loop(0, NG, finish, 0)

    pltpu.sync_copy(out_v, out_hbm.at[pl.ds(base, BPW)])


@functools.partial(
    pl.kernel,
    out_type=jax.ShapeDtypeStruct((B,), jnp.float32),
    mesh=plsc.VectorSubcoreMesh(core_axis_name="c", subcore_axis_name="s"),
    scratch_types=[
        pltpu.VMEM((F, BPW), jnp.int32),
        pltpu.VMEM((D * F, BPW), jnp.int32),
        pltpu.VMEM((D * F, BPW), jnp.float32),
        pltpu.VMEM((F, BPW), jnp.float32),
        pltpu.VMEM((BPW,), jnp.float32),
        pltpu.VMEM((BPW,), jnp.float32),
        pltpu.VMEM((L,), jnp.float32),
        pltpu.SemaphoreType.DMA,
        pltpu.SemaphoreType.DMA,
    ],
    compiler_params=pltpu.CompilerParams(use_tc_tiling_on_sc=False),
)
def _fm_kernel(xt_hbm, embf_hbm, lint_hbm, bias_hbm, out_hbm,
               idxt_v, addr_v, cols_v, lin_v, ix_v, out_v, bias_v,
               sem_e, sem_l):
    _fm_body(xt_hbm, embf_hbm, lint_hbm, bias_hbm, out_hbm,
             idxt_v, addr_v, cols_v, lin_v, ix_v, out_v, bias_v,
             sem_e, sem_l)


def kernel(x, linear_w, emb_w, bias):
    xt = x.astype(jnp.int32).T.reshape(F, NW, BPW)
    embf = _relayout(emb_w.T).reshape(CT * D * 128)
    lint = jnp.pad(linear_w, ((0, VPAD - V), (0, 0))).T.reshape(VPAD)
    bias_vec = jnp.broadcast_to(bias.astype(jnp.float32), (L,))
    out = _fm_kernel(xt, embf, lint, bias_vec)
    return out.reshape(B, 1)


# speedup vs baseline: 7.6837x; 2.4847x over previous
import functools

import jax
import jax.numpy as jnp
from jax import lax
from jax.experimental import pallas as pl
from jax.experimental.pallas import tpu as pltpu
from jax.experimental.pallas import tpu_sc as plsc

NC = 2
NS = 16
NW = NC * NS
L = 16

B = 4096
F = 26
D = 16
V = 1_000_000

BPW = B // NW
NG = BPW // L
CT = 7813
VPAD = CT * 128
RCH = 601


def _relayout_body(in_ref, out_ref):
    for j in range(RCH):
        out_ref[j] = in_ref[:, pl.ds(j * 128, 128)]


_relayout = pl.pallas_call(
    _relayout_body,
    grid=(CT // RCH,),
    in_specs=[pl.BlockSpec((D, RCH * 128), lambda c: (0, c))],
    out_specs=pl.BlockSpec((RCH, D, 128), lambda c: (c, 0, 0)),
    out_shape=jax.ShapeDtypeStruct((CT, D, 128), jnp.float32),
)


def _fm_body(xt_hbm, embf_hbm, lint_hbm, bias_hbm, out_hbm,
             idxt_v, addr_v, cols_v, lin_v, ix_v, out_v, bias_v,
             sem_e, sem_l):
    c = lax.axis_index("c")
    s = lax.axis_index("s")
    wid = s * NC + c
    base = wid * BPW

    def stage(f, carry):
        pltpu.sync_copy(xt_hbm.at[f, wid], idxt_v.at[f])
        return carry

    lax.fori_loop(0, F, stage, 0)
    pltpu.sync_copy(bias_hbm, bias_v)

    def fire_lin(f, carry):
        pltpu.make_async_copy(
            lint_hbm.at[idxt_v.at[f]], lin_v.at[f], sem_l,
        ).start()
        return carry

    lax.fori_loop(0, F, fire_lin, 0)

    def mk_addr(k, carry):
        f = k // (BPW // L)
        j = k - f * (BPW // L)
        sl = pl.ds(j * L, L)
        v = idxt_v[f, sl]
        b = ((v >> 7) << 11) | (v & 127)
        for d in range(D):
            addr_v[d * F + f, sl] = b + (d * 128)
        return carry

    lax.fori_loop(0, F * (BPW // L), mk_addr, 0)

    def fire_emb(k, carry):
        pltpu.make_async_copy(
            embf_hbm.at[addr_v.at[k]], cols_v.at[k], sem_e,
        ).start()
        return carry

    lax.fori_loop(0, D * F, fire_emb, 0)

    def init_ix(g, carry):
        ix_v[pl.ds(g * L, L)] = jnp.zeros((L,), jnp.float32)
        return carry

    lax.fori_loop(0, NG, init_ix, 0)

    def per_d(d, carry):
        def drain(f, c2):
            pltpu.make_async_copy(
                embf_hbm.at[pl.ds(0, BPW)], cols_v.at[d * F + f], sem_e,
            ).wait()
            return c2

        lax.fori_loop(0, F, drain, 0)

        def per_group(g, c2):
            col = pl.ds(g * L, L)
            v = cols_v[d * F, col]
            s_acc = v
            ss_acc = v * v
            for f in range(1, F):
                v = cols_v[d * F + f, col]
                s_acc = s_acc + v
                ss_acc = ss_acc + v * v
            ix_v[col] = ix_v[col] + s_acc * s_acc - ss_acc
            return c2

        lax.fori_loop(0, NG, per_group, 0)
        return carry

    lax.fori_loop(0, D, per_d, 0)

    def drain_lin(f, carry):
        pltpu.make_async_copy(
            lint_hbm.at[pl.ds(0, BPW)], lin_v.at[f], sem_l,
        ).wait()
        return carry

    lax.fori_loop(0, F, drain_lin, 0)
    bias_vec = bias_v[...]

    def finish(g, carry):
        col = pl.ds(g * L, L)
        lin_acc = lin_v[0, col]
        for f in range(1, F):
            lin_acc = lin_acc + lin_v[f, col]
        z = ix_v[col] + lin_acc + bias_vec
        out_v[col] = 1.0 / (1.0 + jnp.exp(-z))
        return carry

    lax.fori_loop(0, NG, finish, 0)

    pltpu.sync_copy(out_v, out_hbm.at[pl.ds(base, BPW)])


@functools.partial(
    pl.kernel,
    out_type=jax.ShapeDtypeStruct((B,), jnp.float32),
    mesh=plsc.VectorSubcoreMesh(core_axis_name="c", subcore_axis_name="s"),
    scratch_types=[
        pltpu.VMEM((F, BPW), jnp.int32),
        pltpu.VMEM((D * F, BPW), jnp.int32),
        pltpu.VMEM((D * F, BPW), jnp.float32),
        pltpu.VMEM((F, BPW), jnp.float32),
        pltpu.VMEM((BPW,), jnp.float32),
        pltpu.VMEM((BPW,), jnp.float32),
        pltpu.VMEM((L,), jnp.float32),
        pltpu.SemaphoreType.DMA,
        pltpu.SemaphoreType.DMA,
    ],
    compiler_params=pltpu.CompilerParams(use_tc_tiling_on_sc=False),
)
def _fm_kernel(xt_hbm, embf_hbm, lint_hbm, bias_hbm, out_hbm,
               idxt_v, addr_v, cols_v, lin_v, ix_v, out_v, bias_v,
               sem_e, sem_l):
    _fm_body(xt_hbm, embf_hbm, lint_hbm, bias_hbm, out_hbm,
             idxt_v, addr_v, cols_v, lin_v, ix_v, out_v, bias_v,
             sem_e, sem_l)


def kernel(x, linear_w, emb_w, bias):
    xt = x.astype(jnp.int32).T.reshape(F, NW, BPW)
    embf = _relayout(emb_w.T).reshape(CT * D * 128)
    lint = jnp.pad(linear_w, ((0, VPAD - V), (0, 0))).T.reshape(VPAD)
    bias_vec = jnp.broadcast_to(bias.astype(jnp.float32), (L,))
    out = _fm_kernel(xt, embf, lint, bias_vec)
    return out.reshape(B, 1)


# speedup vs baseline: 7.7252x; 1.0054x over previous
import functools

import jax
import jax.numpy as jnp
from jax import lax
from jax.experimental import pallas as pl
from jax.experimental.pallas import tpu as pltpu
from jax.experimental.pallas import tpu_sc as plsc

NC = 2
NS = 16
NW = NC * NS
L = 16

B = 4096
F = 26
D = 16
V = 1_000_000

BPW = B // NW
NG = BPW // L
CT = 7813
VPAD = CT * 128
RCH = 601


def _relayout_body(emb_ref, lin_ref, oute_ref, outl_ref):
    for j in range(RCH):
        oute_ref[j] = emb_ref[:, pl.ds(j * 128, 128)]
    outl_ref[0] = lin_ref[...]


_relayout = pl.pallas_call(
    _relayout_body,
    grid=(CT // RCH,),
    in_specs=[
        pl.BlockSpec((D, RCH * 128), lambda c: (0, c)),
        pl.BlockSpec((1, RCH * 128), lambda c: (0, c)),
    ],
    out_specs=[
        pl.BlockSpec((RCH, D, 128), lambda c: (c, 0, 0)),
        pl.BlockSpec((1, 1, RCH * 128), lambda c: (c, 0, 0)),
    ],
    out_shape=[
        jax.ShapeDtypeStruct((CT, D, 128), jnp.float32),
        jax.ShapeDtypeStruct((CT // RCH, 1, RCH * 128), jnp.float32),
    ],
)


def _fm_body(xt_hbm, embf_hbm, lint_hbm, bias_hbm, out_hbm,
             idxt_v, addr_v, cols_v, lin_v, ix_v, out_v, bias_v,
             sem_e, sem_l):
    c = lax.axis_index("c")
    s = lax.axis_index("s")
    wid = s * NC + c
    base = wid * BPW

    def stage(f, carry):
        pltpu.sync_copy(xt_hbm.at[f, wid], idxt_v.at[f])
        return carry

    lax.fori_loop(0, F, stage, 0)
    pltpu.sync_copy(bias_hbm, bias_v)

    def fire_lin(f, carry):
        pltpu.make_async_copy(
            lint_hbm.at[idxt_v.at[f]], lin_v.at[f], sem_l,
        ).start()
        return carry

    lax.fori_loop(0, F, fire_lin, 0)

    def mk_addr(k, carry):
        f = k // (BPW // L)
        j = k - f * (BPW // L)
        sl = pl.ds(j * L, L)
        v = idxt_v[f, sl]
        b = ((v >> 7) << 11) | (v & 127)
        for d in range(D):
            addr_v[d * F + f, sl] = b + (d * 128)
        return carry

    lax.fori_loop(0, F * (BPW // L), mk_addr, 0)

    def fire_emb(k, carry):
        pltpu.make_async_copy(
            embf_hbm.at[addr_v.at[k]], cols_v.at[k], sem_e,
        ).start()
        return carry

    lax.fori_loop(0, D * F, fire_emb, 0)

    def init_ix(g, carry):
        ix_v[pl.ds(g * L, L)] = jnp.zeros((L,), jnp.float32)
        return carry

    lax.fori_loop(0, NG, init_ix, 0)

    def per_d(d, carry):
        def drain(f, c2):
            pltpu.make_async_copy(
                embf_hbm.at[pl.ds(0, BPW)], cols_v.at[d * F + f], sem_e,
            ).wait()
            return c2

        lax.fori_loop(0, F, drain, 0)

        def per_group(g, c2):
            col = pl.ds(g * L, L)
            v = cols_v[d * F, col]
            s_acc = v
            ss_acc = v * v
            for f in range(1, F):
                v = cols_v[d * F + f, col]
                s_acc = s_acc + v
                ss_acc = ss_acc + v * v
            ix_v[col] = ix_v[col] + s_acc * s_acc - ss_acc
            return c2

        lax.fori_loop(0, NG, per_group, 0)
        return carry

    lax.fori_loop(0, D, per_d, 0)

    def drain_lin(f, carry):
        pltpu.make_async_copy(
            lint_hbm.at[pl.ds(0, BPW)], lin_v.at[f], sem_l,
        ).wait()
        return carry

    lax.fori_loop(0, F, drain_lin, 0)
    bias_vec = bias_v[...]

    def finish(g, carry):
        col = pl.ds(g * L, L)
        lin_acc = lin_v[0, col]
        for f in range(1, F):
            lin_acc = lin_acc + lin_v[f, col]
        z = ix_v[col] + lin_acc + bias_vec
        out_v[col] = 1.0 / (1.0 + jnp.exp(-z))
        return carry

    lax.fori_loop(0, NG, finish, 0)

    pltpu.sync_copy(out_v, out_hbm.at[pl.ds(base, BPW)])


@functools.partial(
    pl.kernel,
    out_type=jax.ShapeDtypeStruct((B,), jnp.float32),
    mesh=plsc.VectorSubcoreMesh(core_axis_name="c", subcore_axis_name="s"),
    scratch_types=[
        pltpu.VMEM((F, BPW), jnp.int32),
        pltpu.VMEM((D * F, BPW), jnp.int32),
        pltpu.VMEM((D * F, BPW), jnp.float32),
        pltpu.VMEM((F, BPW), jnp.float32),
        pltpu.VMEM((BPW,), jnp.float32),
        pltpu.VMEM((BPW,), jnp.float32),
        pltpu.VMEM((L,), jnp.float32),
        pltpu.SemaphoreType.DMA,
        pltpu.SemaphoreType.DMA,
    ],
    compiler_params=pltpu.CompilerParams(use_tc_tiling_on_sc=False),
)
def _fm_kernel(xt_hbm, embf_hbm, lint_hbm, bias_hbm, out_hbm,
               idxt_v, addr_v, cols_v, lin_v, ix_v, out_v, bias_v,
               sem_e, sem_l):
    _fm_body(xt_hbm, embf_hbm, lint_hbm, bias_hbm, out_hbm,
             idxt_v, addr_v, cols_v, lin_v, ix_v, out_v, bias_v,
             sem_e, sem_l)


def kernel(x, linear_w, emb_w, bias):
    xt = x.astype(jnp.int32).T.reshape(F, NW, BPW)
    embf, lin3 = _relayout(emb_w.T, linear_w.T)
    embf = embf.reshape(CT * D * 128)
    lint = lin3.reshape(VPAD)
    bias_vec = jnp.broadcast_to(bias.astype(jnp.float32), (L,))
    out = _fm_kernel(xt, embf, lint, bias_vec)
    return out.reshape(B, 1)


# speedup vs baseline: 7.8457x; 1.0156x over previous
import functools

import jax
import jax.numpy as jnp
from jax import lax
from jax.experimental import pallas as pl
from jax.experimental.pallas import tpu as pltpu
from jax.experimental.pallas import tpu_sc as plsc

NC = 2
NS = 16
NW = NC * NS
L = 16

B = 4096
F = 26
D = 16
V = 1_000_000

BPW = B // NW
NG = BPW // L
CT = 7813
VPAD = CT * 128
RCH = 601


def _relayout_body(emb_ref, lin_ref, oute_ref, outl_ref):
    for j in range(RCH):
        oute_ref[j] = emb_ref[:, pl.ds(j * 128, 128)]
    outl_ref[...] = lin_ref[...]


_relayout = pl.pallas_call(
    _relayout_body,
    grid=(CT // RCH,),
    in_specs=[
        pl.BlockSpec((D, RCH * 128), lambda c: (0, c)),
        pl.BlockSpec((1, RCH * 128), lambda c: (0, c)),
    ],
    out_specs=[
        pl.BlockSpec((RCH, D, 128), lambda c: (c, 0, 0)),
        pl.BlockSpec((1, RCH * 128), lambda c: (0, c)),
    ],
    out_shape=[
        jax.ShapeDtypeStruct((CT, D, 128), jnp.float32),
        jax.ShapeDtypeStruct((1, VPAD), jnp.float32),
    ],
)


def _fm_body(xt_hbm, embf_hbm, lint_hbm, bias_hbm, out_hbm,
             idxt_v, addr_v, cols_v, lin_v, ix_v, out_v, bias_v,
             sem_e, sem_l):
    c = lax.axis_index("c")
    s = lax.axis_index("s")
    wid = s * NC + c
    base = wid * BPW

    def stage(f, carry):
        pltpu.sync_copy(xt_hbm.at[f, wid], idxt_v.at[f])
        return carry

    lax.fori_loop(0, F, stage, 0)
    pltpu.sync_copy(bias_hbm, bias_v)

    def fire_lin(f, carry):
        pltpu.make_async_copy(
            lint_hbm.at[idxt_v.at[f]], lin_v.at[f], sem_l,
        ).start()
        return carry

    lax.fori_loop(0, F, fire_lin, 0)

    def mk_addr(k, carry):
        f = k // (BPW // L)
        j = k - f * (BPW // L)
        sl = pl.ds(j * L, L)
        v = idxt_v[f, sl]
        b = ((v >> 7) << 11) | (v & 127)
        for d in range(D):
            addr_v[d * F + f, sl] = b + (d * 128)
        return carry

    lax.fori_loop(0, F * (BPW // L), mk_addr, 0)

    def fire_emb(k, carry):
        pltpu.make_async_copy(
            embf_hbm.at[addr_v.at[k]], cols_v.at[k], sem_e,
        ).start()
        return carry

    lax.fori_loop(0, D * F, fire_emb, 0)

    def init_ix(g, carry):
        ix_v[pl.ds(g * L, L)] = jnp.zeros((L,), jnp.float32)
        return carry

    lax.fori_loop(0, NG, init_ix, 0)

    def per_d(d, carry):
        def drain(f, c2):
            pltpu.make_async_copy(
                embf_hbm.at[pl.ds(0, BPW)], cols_v.at[d * F + f], sem_e,
            ).wait()
            return c2

        lax.fori_loop(0, F, drain, 0)

        def per_group(g, c2):
            col = pl.ds(g * L, L)
            v = cols_v[d * F, col]
            s_acc = v
            ss_acc = v * v
            for f in range(1, F):
                v = cols_v[d * F + f, col]
                s_acc = s_acc + v
                ss_acc = ss_acc + v * v
            ix_v[col] = ix_v[col] + s_acc * s_acc - ss_acc
            return c2

        lax.fori_loop(0, NG, per_group, 0)
        return carry

    lax.fori_loop(0, D, per_d, 0)

    def drain_lin(f, carry):
        pltpu.make_async_copy(
            lint_hbm.at[pl.ds(0, BPW)], lin_v.at[f], sem_l,
        ).wait()
        return carry

    lax.fori_loop(0, F, drain_lin, 0)
    bias_vec = bias_v[...]

    def finish(g, carry):
        col = pl.ds(g * L, L)
        lin_acc = lin_v[0, col]
        for f in range(1, F):
            lin_acc = lin_acc + lin_v[f, col]
        z = ix_v[col] + lin_acc + bias_vec
        out_v[col] = 1.0 / (1.0 + jnp.exp(-z))
        return carry

    lax.fori_loop(0, NG, finish, 0)

    pltpu.sync_copy(out_v, out_hbm.at[pl.ds(base, BPW)])


@functools.partial(
    pl.kernel,
    out_type=jax.ShapeDtypeStruct((B,), jnp.float32),
    mesh=plsc.VectorSubcoreMesh(core_axis_name="c", subcore_axis_name="s"),
    scratch_types=[
        pltpu.VMEM((F, BPW), jnp.int32),
        pltpu.VMEM((D * F, BPW), jnp.int32),
        pltpu.VMEM((D * F, BPW), jnp.float32),
        pltpu.VMEM((F, BPW), jnp.float32),
        pltpu.VMEM((BPW,), jnp.float32),
        pltpu.VMEM((BPW,), jnp.float32),
        pltpu.VMEM((L,), jnp.float32),
        pltpu.SemaphoreType.DMA,
        pltpu.SemaphoreType.DMA,
    ],
    compiler_params=pltpu.CompilerParams(use_tc_tiling_on_sc=False),
)
def _fm_kernel(xt_hbm, embf_hbm, lint_hbm, bias_hbm, out_hbm,
               idxt_v, addr_v, cols_v, lin_v, ix_v, out_v, bias_v,
               sem_e, sem_l):
    _fm_body(xt_hbm, embf_hbm, lint_hbm, bias_hbm, out_hbm,
             idxt_v, addr_v, cols_v, lin_v, ix_v, out_v, bias_v,
             sem_e, sem_l)


def kernel(x, linear_w, emb_w, bias):
    xt = x.astype(jnp.int32).T.reshape(F, NW, BPW)
    embf, lin3 = _relayout(emb_w.T, linear_w.T)
    embf = embf.reshape(CT * D * 128)
    lint = lin3.reshape(VPAD)
    bias_vec = jnp.broadcast_to(bias.astype(jnp.float32), (L,))
    out = _fm_kernel(xt, embf, lint, bias_vec)
    return out.reshape(B, 1)


# speedup vs baseline: 9.3696x; 1.1942x over previous
import functools

import jax
import jax.numpy as jnp
from jax import lax
from jax.experimental import pallas as pl
from jax.experimental.pallas import tpu as pltpu
from jax.experimental.pallas import tpu_sc as plsc

NC = 2
NS = 16
NW = NC * NS
L = 16

B = 4096
F = 26
D = 16
V = 1_000_000

BPW = B // NW
NG = BPW // L
CT = 7813
VPAD = CT * 128
RCH = 601


def _relayout_body(emb_ref, lin_ref, oute_ref, outl_ref):
    for j in range(RCH):
        oute_ref[j] = emb_ref[:, pl.ds(j * 128, 128)]
        outl_ref[j, 0] = lin_ref[0, pl.ds(j * 128, 128)]


_relayout = pl.pallas_call(
    _relayout_body,
    grid=(CT // RCH,),
    in_specs=[
        pl.BlockSpec((D, RCH * 128), lambda c: (0, c)),
        pl.BlockSpec((1, RCH * 128), lambda c: (0, c)),
    ],
    out_specs=[
        pl.BlockSpec((RCH, D, 128), lambda c: (c, 0, 0)),
        pl.BlockSpec((RCH, 8, 128), lambda c: (c, 0, 0)),
    ],
    out_shape=[
        jax.ShapeDtypeStruct((CT, D, 128), jnp.float32),
        jax.ShapeDtypeStruct((CT, 8, 128), jnp.float32),
    ],
)


def _fm_body(xt_hbm, embf_hbm, lint_hbm, bias_hbm, out_hbm,
             idxt_v, ladr_v, addr_v, cols_v, lin_v, ix_v, out_v, bias_v,
             sem_e, sem_l):
    c = lax.axis_index("c")
    s = lax.axis_index("s")
    wid = s * NC + c
    base = wid * BPW

    def stage(f, carry):
        pltpu.sync_copy(xt_hbm.at[f, wid], idxt_v.at[f])
        return carry

    lax.fori_loop(0, F, stage, 0)
    pltpu.sync_copy(bias_hbm, bias_v)

    def fire_lin(f, carry):
        pltpu.make_async_copy(
            lint_hbm.at[ladr_v.at[f]], lin_v.at[f], sem_l,
        ).start()
        return carry

    lax.fori_loop(0, F, fire_lin, 0)

    def mk_addr(k, carry):
        f = k // (BPW // L)
        j = k - f * (BPW // L)
        sl = pl.ds(j * L, L)
        v = idxt_v[f, sl]
        b = ((v >> 7) << 11) | (v & 127)
        ladr_v[f, sl] = ((v >> 7) << 10) | (v & 127)
        for d in range(D):
            addr_v[d * F + f, sl] = b + (d * 128)
        return carry

    lax.fori_loop(0, F * (BPW // L), mk_addr, 0)

    def fire_emb(k, carry):
        pltpu.make_async_copy(
            embf_hbm.at[addr_v.at[k]], cols_v.at[k], sem_e,
        ).start()
        return carry

    lax.fori_loop(0, D * F, fire_emb, 0)

    def init_ix(g, carry):
        ix_v[pl.ds(g * L, L)] = jnp.zeros((L,), jnp.float32)
        return carry

    lax.fori_loop(0, NG, init_ix, 0)

    def per_d(d, carry):
        def drain(f, c2):
            pltpu.make_async_copy(
                embf_hbm.at[pl.ds(0, BPW)], cols_v.at[d * F + f], sem_e,
            ).wait()
            return c2

        lax.fori_loop(0, F, drain, 0)

        def per_group(g, c2):
            col = pl.ds(g * L, L)
            v = cols_v[d * F, col]
            s_acc = v
            ss_acc = v * v
            for f in range(1, F):
                v = cols_v[d * F + f, col]
                s_acc = s_acc + v
                ss_acc = ss_acc + v * v
            ix_v[col] = ix_v[col] + s_acc * s_acc - ss_acc
            return c2

        lax.fori_loop(0, NG, per_group, 0)
        return carry

    lax.fori_loop(0, D, per_d, 0)

    def drain_lin(f, carry):
        pltpu.make_async_copy(
            lint_hbm.at[pl.ds(0, BPW)], lin_v.at[f], sem_l,
        ).wait()
        return carry

    lax.fori_loop(0, F, drain_lin, 0)
    bias_vec = bias_v[...]

    def finish(g, carry):
        col = pl.ds(g * L, L)
        lin_acc = lin_v[0, col]
        for f in range(1, F):
            lin_acc = lin_acc + lin_v[f, col]
        z = ix_v[col] + lin_acc + bias_vec
        out_v[col] = 1.0 / (1.0 + jnp.exp(-z))
        return carry

    lax.fori_loop(0, NG, finish, 0)

    pltpu.sync_copy(out_v, out_hbm.at[pl.ds(base, BPW)])


@functools.partial(
    pl.kernel,
    out_type=jax.ShapeDtypeStruct((B,), jnp.float32),
    mesh=plsc.VectorSubcoreMesh(core_axis_name="c", subcore_axis_name="s"),
    scratch_types=[
        pltpu.VMEM((F, BPW), jnp.int32),
        pltpu.VMEM((F, BPW), jnp.int32),
        pltpu.VMEM((D * F, BPW), jnp.int32),
        pltpu.VMEM((D * F, BPW), jnp.float32),
        pltpu.VMEM((F, BPW), jnp.float32),
        pltpu.VMEM((BPW,), jnp.float32),
        pltpu.VMEM((BPW,), jnp.float32),
        pltpu.VMEM((L,), jnp.float32),
        pltpu.SemaphoreType.DMA,
        pltpu.SemaphoreType.DMA,
    ],
    compiler_params=pltpu.CompilerParams(use_tc_tiling_on_sc=False),
)
def _fm_kernel(xt_hbm, embf_hbm, lint_hbm, bias_hbm, out_hbm,
               idxt_v, ladr_v, addr_v, cols_v, lin_v, ix_v, out_v, bias_v,
               sem_e, sem_l):
    _fm_body(xt_hbm, embf_hbm, lint_hbm, bias_hbm, out_hbm,
             idxt_v, ladr_v, addr_v, cols_v, lin_v, ix_v, out_v, bias_v,
             sem_e, sem_l)


def kernel(x, linear_w, emb_w, bias):
    xt = x.astype(jnp.int32).T.reshape(F, NW, BPW)
    embf, lin3 = _relayout(emb_w.T, linear_w.T)
    embf = embf.reshape(CT * D * 128)
    lint = lin3.reshape(CT * 8 * 128)
    bias_vec = jnp.broadcast_to(bias.astype(jnp.float32), (L,))
    out = _fm_kernel(xt, embf, lint, bias_vec)
    return out.reshape(B, 1)
